# Initial kernel scaffold; baseline (speedup 1.0000x reference)
import jax
import jax.numpy as jnp
from jax.experimental import pallas as pl


def kernel(X, lengths):
    raise NotImplementedError("write your pallas kernel here")


# speedup vs baseline: 1.4765x; 1.4765x over previous
import jax
import jax.numpy as jnp
from jax.experimental import pallas as pl
from jax.experimental.pallas import tpu as pltpu

_A = 1280
_FR = 0.15
_TR = 0.2

_ROW_BLK = 256


def _mask_bounds(lengths):
    B = lengths.shape[0]
    Ti = lengths.astype(jnp.int32)
    key = jax.random.key(42)
    key, ka, kb = jax.random.split(key, 3)
    max_t = jnp.maximum(1, jnp.floor(Ti.astype(jnp.float32) * _TR).astype(jnp.int32))
    u = jax.random.uniform(ka, (B,))
    t = 1 + jnp.floor(u * max_t.astype(jnp.float32)).astype(jnp.int32)
    t = jnp.minimum(t, max_t)
    room = Ti - t
    u2 = jax.random.uniform(kb, (B,))
    t0 = jnp.where(room > 0,
                   jnp.floor(u2 * (room + 1).astype(jnp.float32)).astype(jnp.int32),
                   0)
    valid = Ti > 0
    big = jnp.int32(1 << 30)
    tlo = jnp.where(valid, t0, big)
    thi = jnp.where(valid, t0 + t, big)
    max_f = int(_A * _FR)
    key, ka2, kb2 = jax.random.split(key, 3)
    f = jax.random.randint(ka2, (B,), 1, max_f + 1)
    f0_max = jnp.clip(_A - f, 0, None)
    f0 = jnp.floor(jax.random.uniform(kb2, (B,)) * (f0_max + 1).astype(jnp.float32)
                   ).astype(jnp.int32)
    return tlo, thi, f0, f0 + f


def _tc_body(s_ref, x_ref, o_ref):
    b = pl.program_id(0)
    tb = pl.program_id(1)
    tlo = s_ref[b, 0]
    thi = s_ref[b, 1]
    flo = s_ref[b, 2]
    fhi = s_ref[b, 3]
    x = x_ref[0]
    rows = jax.lax.broadcasted_iota(jnp.int32, x.shape, 0) + tb * _ROW_BLK
    cols = jax.lax.broadcasted_iota(jnp.int32, x.shape, 1)
    in_t = (rows >= tlo) & (rows < thi)
    in_f = (cols >= flo) & (cols < fhi)
    audio = cols >= s_ref[0, 4]
    mask = audio & (in_t | in_f)
    o_ref[0] = jnp.where(mask, 0.0, x)


def kernel(X, lengths):
    B, T, D = X.shape
    off = D - _A
    tlo, thi, flo, fhi = _mask_bounds(lengths)
    s = jnp.stack(
        [tlo, thi, flo + off, fhi + off, jnp.full_like(tlo, off)], axis=1)
    return pl.pallas_call(
        _tc_body,
        grid=(B, T // _ROW_BLK),
        in_specs=[
            pl.BlockSpec(memory_space=pltpu.SMEM),
            pl.BlockSpec((1, _ROW_BLK, D), lambda b, t: (b, t, 0)),
        ],
        out_specs=pl.BlockSpec((1, _ROW_BLK, D), lambda b, t: (b, t, 0)),
        out_shape=jax.ShapeDtypeStruct((B, T, D), X.dtype),
        compiler_params=pltpu.CompilerParams(
            dimension_semantics=("parallel", "parallel")),
    )(s, X)
